# Initial kernel scaffold; baseline (speedup 1.0000x reference)
import jax
import jax.numpy as jnp
from jax.experimental import pallas as pl


def kernel(lesion_hidden, lesion_boxes, lesion_logits, box_W1, box_b1, box_W2, box_b2, sc_W1, sc_b1, sc_W2, sc_b2, ln_g, ln_b):
    raise NotImplementedError("write your pallas kernel here")


# speedup vs baseline: 1.6570x; 1.6570x over previous
import functools

import jax
import jax.numpy as jnp
from jax import lax
from jax.experimental import pallas as pl
from jax.experimental.pallas import tpu as pltpu
from jax.experimental.pallas import tpu_sc as plsc

_B, _N, _D, _C, _K = 128, 900, 256, 81, 300
_NW = 32
_ROWS_PER_W = (_B * _K) // _NW
_BPW = _ROWS_PER_W // _K
_NP = 1024
_R = 8


def _topk_body(logits_ref, scores_ref, idx_ref, flat_ref, lflat_ref):
    step = pl.program_id(0)
    pos = (lax.broadcasted_iota(jnp.int32, (8, 128), 0) * 128
           + lax.broadcasted_iota(jnp.int32, (8, 128), 1))
    for r in range(_R):
        x = logits_ref[r]
        lane = lax.broadcasted_iota(jnp.int32, (_N, _C), 1)
        x = jnp.where(lane < _C - 1, x, -jnp.inf)
        m2 = jnp.max(x, axis=1, keepdims=True)
        mp = jnp.concatenate(
            [m2, jnp.full((_NP - _N, 1), -jnp.inf, jnp.float32)], axis=0)
        z = mp.reshape(8, 128)
        u = lax.bitcast_convert_type(z, jnp.uint32)
        key = jnp.where((u >> 31) != 0, ~u, u | jnp.uint32(0x80000000))
        idx = pos

        for kk_log in range(1, 11):
            kk = 1 << kk_log
            gf = (pos & kk) == 0
            for d_log in range(kk_log - 1, -1, -1):
                d = 1 << d_log
                bit = (pos & d) != 0
                if d < 128:
                    rk_p, rk_m = pltpu.roll(key, d, 1), pltpu.roll(key, 128 - d, 1)
                    ri_p, ri_m = pltpu.roll(idx, d, 1), pltpu.roll(idx, 128 - d, 1)
                else:
                    sd = d // 128
                    rk_p, rk_m = pltpu.roll(key, sd, 0), pltpu.roll(key, 8 - sd, 0)
                    ri_p, ri_m = pltpu.roll(idx, sd, 0), pltpu.roll(idx, 8 - sd, 0)
                pk = jnp.where(bit, rk_p, rk_m)
                pi = jnp.where(bit, ri_p, ri_m)
                p_first = (pk > key) | ((pk == key) & (pi < idx))
                take = ~(bit ^ gf ^ p_first)
                key = jnp.where(take, pk, key)
                idx = jnp.where(take, pi, idx)

        um = jnp.where((key >> 31) != 0, key & jnp.uint32(0x7FFFFFFF), ~key)
        ms = lax.bitcast_convert_type(um, jnp.float32)
        scores_ref[r] = jax.nn.sigmoid(ms)
        idx_ref[r] = idx
        b = step * _R + r
        flat_ref[r] = idx + b * _N
        lflat_ref[r] = idx + (b % _BPW) * _N


def _run_topk(lesion_logits):
    nsteps = _B // _R
    return pl.pallas_call(
        _topk_body,
        grid=(nsteps,),
        in_specs=[pl.BlockSpec((_R, _N, _C), lambda i: (i, 0, 0))],
        out_specs=[
            pl.BlockSpec((_R, 8, 128), lambda i: (i, 0, 0)),
            pl.BlockSpec((_R, 8, 128), lambda i: (i, 0, 0)),
            pl.BlockSpec((_R, 8, 128), lambda i: (i, 0, 0)),
            pl.BlockSpec((_R, 8, 128), lambda i: (i, 0, 0)),
        ],
        out_shape=[
            jax.ShapeDtypeStruct((_B, 8, 128), jnp.float32),
            jax.ShapeDtypeStruct((_B, 8, 128), jnp.int32),
            jax.ShapeDtypeStruct((_B, 8, 128), jnp.int32),
            jax.ShapeDtypeStruct((_B, 8, 128), jnp.int32),
        ],
    )(lesion_logits)


_CHUNK = 120
_SRC_PER_W = _BPW * _N


def _sc_gather(hidden_flat, boxes_flat, flat_idx, local_idx):
    mesh = plsc.VectorSubcoreMesh(core_axis_name="c", subcore_axis_name="s")

    @functools.partial(
        pl.kernel,
        out_type=[
            jax.ShapeDtypeStruct((_B * _K, _D), jnp.float32),
            jax.ShapeDtypeStruct((_B * _K * 4,), jnp.float32),
        ],
        mesh=mesh,
        compiler_params=pltpu.CompilerParams(needs_layout_passes=False),
        scratch_types=[
            pltpu.VMEM((_ROWS_PER_W,), jnp.int32),
            pltpu.VMEM((_ROWS_PER_W,), jnp.int32),
            pltpu.VMEM((_CHUNK, _D), jnp.float32),
            pltpu.VMEM((_SRC_PER_W * 4,), jnp.float32),
            pltpu.VMEM((_ROWS_PER_W * 4,), jnp.float32),
            pltpu.SemaphoreType.DMA,
        ],
    )
    def k(hid_hbm, box_hbm, idx_hbm, lidx_hbm, out_hbm, boxout_hbm,
          idx_v, lidx_v, rows_v, boxsrc_v, boxdst_v, sem):
        nc = 2
        wid = lax.axis_index("s") * nc + lax.axis_index("c")
        base = wid * _ROWS_PER_W
        pltpu.sync_copy(idx_hbm.at[pl.ds(base, _ROWS_PER_W)], idx_v)
        pltpu.sync_copy(lidx_hbm.at[pl.ds(base, _ROWS_PER_W)], lidx_v)
        pltpu.sync_copy(box_hbm.at[pl.ds(wid * _SRC_PER_W * 4, _SRC_PER_W * 4)],
                        boxsrc_v)
        for c in range(_ROWS_PER_W // _CHUNK):
            idx_c = idx_v.at[pl.ds(c * _CHUNK, _CHUNK)]
            pltpu.async_copy(hid_hbm.at[idx_c], rows_v, sem).wait()
            pltpu.sync_copy(rows_v, out_hbm.at[pl.ds(base + c * _CHUNK, _CHUNK)])
        lane = lax.iota(jnp.int32, 16)
        for g in range(_ROWS_PER_W // 16):
            idx16 = lidx_v[pl.ds(g * 16, 16)] * 4
            pos16 = (lane + g * 16) * 4
            for col in range(4):
                vals = plsc.load_gather(boxsrc_v, [idx16 + col])
                plsc.store_scatter(boxdst_v, [pos16 + col], vals)
        pltpu.sync_copy(boxdst_v, boxout_hbm.at[pl.ds(base * 4, _ROWS_PER_W * 4)])

    return k(hidden_flat, boxes_flat, flat_idx, local_idx)


_BLK = 768


def _gelu_exact(x):
    return x * 0.5 * (1.0 + lax.erf(x * 0.7071067811865476))


def _mlp_body(hid_ref, box_ref, sc_ref, w1_ref, b1_ref, scw1_ref, scb1_ref,
              wcat_ref, bsum_ref, g_ref, beta_ref, out_ref):
    h_box = jnp.dot(box_ref[...], w1_ref[...],
                    preferred_element_type=jnp.float32) + b1_ref[...]
    h_box = _gelu_exact(h_box)
    h_sc = _gelu_exact(sc_ref[...] * scw1_ref[...] + scb1_ref[...])
    hcat = jnp.concatenate([h_box, h_sc], axis=1).astype(jnp.bfloat16)
    t = jnp.dot(hcat, wcat_ref[...], preferred_element_type=jnp.float32)
    tok = hid_ref[...] + t + bsum_ref[...]
    mu = jnp.mean(tok, axis=1, keepdims=True)
    var = jnp.mean((tok - mu) ** 2, axis=1, keepdims=True)
    out_ref[...] = ((tok - mu) * lax.rsqrt(var + 1e-5) * g_ref[...]
                    + beta_ref[...])


def _run_mlp(hid, boxg, scores, box_W1, box_b1, sc_W1, sc_b1, wcat, bsum,
             ln_g, ln_b):
    nsteps = (_B * _K) // _BLK
    row = lambda i: (i, 0)
    rep = lambda i: (0, 0)
    return pl.pallas_call(
        _mlp_body,
        grid=(nsteps,),
        in_specs=[
            pl.BlockSpec((_BLK, _D), row),
            pl.BlockSpec((_BLK, 4), row),
            pl.BlockSpec((_BLK, 1), row),
            pl.BlockSpec((4, _D), rep),
            pl.BlockSpec((1, _D), rep),
            pl.BlockSpec((1, _D), rep),
            pl.BlockSpec((1, _D), rep),
            pl.BlockSpec((2 * _D, _D), rep),
            pl.BlockSpec((1, _D), rep),
            pl.BlockSpec((1, _D), rep),
            pl.BlockSpec((1, _D), rep),
        ],
        out_specs=pl.BlockSpec((_BLK, _D), row),
        out_shape=jax.ShapeDtypeStruct((_B * _K, _D), jnp.float32),
    )(hid, boxg, scores, box_W1, box_b1, sc_W1, sc_b1, wcat, bsum, ln_g, ln_b)


def kernel(lesion_hidden, lesion_boxes, lesion_logits, box_W1, box_b1,
           box_W2, box_b2, sc_W1, sc_b1, sc_W2, sc_b2, ln_g, ln_b):
    scores_s, idx_s, flat_s, lflat_s = _run_topk(lesion_logits)
    topk_scores = scores_s.reshape(_B, _NP)[:, :_K]
    topk_indices = idx_s.reshape(_B, _NP)[:, :_K]
    flat_idx = flat_s.reshape(_B, _NP)[:, :_K].reshape(_B * _K)
    local_idx = lflat_s.reshape(_B, _NP)[:, :_K].reshape(_B * _K)

    hidden_flat = lesion_hidden.reshape(_B * _N, _D)
    boxes_flat = lesion_boxes.reshape(_B * _N * 4)
    gathered, boxg4 = _sc_gather(hidden_flat, boxes_flat, flat_idx, local_idx)
    boxg = boxg4.reshape(_B * _K, 4)

    wcat = jnp.concatenate([box_W2, sc_W2], axis=0).astype(jnp.bfloat16)
    bsum = (box_b2 + sc_b2).reshape(1, _D)
    out_flat = _run_mlp(
        gathered, boxg, topk_scores.reshape(_B * _K, 1),
        box_W1, box_b1.reshape(1, _D), sc_W1, sc_b1.reshape(1, _D),
        wcat, bsum, ln_g.reshape(1, _D), ln_b.reshape(1, _D))
    out = out_flat.reshape(_B, _K, _D)
    return (out, topk_scores, topk_indices)
